# Initial kernel scaffold; baseline (speedup 1.0000x reference)
import jax
import jax.numpy as jnp
from jax.experimental import pallas as pl


def kernel(z, codebook):
    raise NotImplementedError("write your pallas kernel here")


# speedup vs baseline: 1.4108x; 1.4108x over previous
import functools

import jax
import jax.numpy as jnp
from jax import lax
from jax.experimental import pallas as pl
from jax.experimental.pallas import tpu as pltpu
from jax.experimental.pallas import tpu_sc as plsc

N_CODES = 8192
DIM = 32
N_TOK = 16384
TOK_BLK = 256
CHUNK = 4096

SC_CORES = 2
SC_SUBCORES = 16
SC_LANES = 16
SC_TILES = SC_CORES * SC_SUBCORES
TOK_PER_TILE = N_TOK // SC_TILES
CNT_PER_SUB = N_CODES // SC_SUBCORES


def _argmin_body(zn_ref, z2bf_ref, cbt_ref, cbtbf_ref, idx_ref):
    zn = zn_ref[...]
    zb = z2bf_ref[...]
    acc_v = jnp.zeros((TOK_BLK, 1), jnp.float32)
    acc_i = jnp.zeros((TOK_BLK, 1), jnp.int32)
    for j in range(N_CODES // CHUNK):
        cb = cbt_ref[:, j * CHUNK:(j + 1) * CHUNK]
        cbbf = cbtbf_ref[:, j * CHUNK:(j + 1) * CHUNK]
        cn = jnp.sum(cb * cb, axis=0, keepdims=True)
        zc2 = lax.dot_general(zb, cbbf, (((1,), (0,)), ((), ())),
                              preferred_element_type=jnp.float32)
        d = (zn + cn) - zc2
        m = jnp.min(d, axis=1, keepdims=True)
        gi = lax.broadcasted_iota(jnp.int32, (TOK_BLK, CHUNK), 1) + (j * CHUNK)
        cand = jnp.where(d == m, gi, jnp.int32(2**30))
        i = jnp.min(cand, axis=1, keepdims=True)
        m_bf = m.astype(jnp.bfloat16).astype(jnp.float32)
        if j == 0:
            acc_v, acc_i = m_bf, i
        else:
            repl = m < acc_v
            acc_v = jnp.where(repl, m_bf, acc_v)
            acc_i = jnp.where(repl, i, acc_i)
    idx_ref[...] = acc_i


def _compute_indices(zn2d, z2bf, cbt, cbt_bf):
    return pl.pallas_call(
        _argmin_body,
        grid=(N_TOK // TOK_BLK,),
        in_specs=[
            pl.BlockSpec((TOK_BLK, 1), lambda i: (i, 0)),
            pl.BlockSpec((TOK_BLK, DIM), lambda i: (i, 0)),
            pl.BlockSpec((DIM, N_CODES), lambda i: (0, 0)),
            pl.BlockSpec((DIM, N_CODES), lambda i: (0, 0)),
        ],
        out_specs=pl.BlockSpec((TOK_BLK, 1), lambda i: (i, 0)),
        out_shape=jax.ShapeDtypeStruct((N_TOK, 1), jnp.int32),
    )(zn2d, z2bf, cbt, cbt_bf)


def _sc_body(cb_hbm, idx_hbm, zq_hbm, counts_hbm,
             idx_v, rows_v, ones_v, zeros_v, shared_counts, sem):
    cid = lax.axis_index("c")
    sid = lax.axis_index("s")
    wid = sid * SC_CORES + cid
    base = wid * TOK_PER_TILE
    pltpu.sync_copy(idx_hbm.at[pl.ds(base, TOK_PER_TILE)], idx_v)
    pltpu.async_copy(cb_hbm.at[idx_v], rows_v, sem).wait()
    pltpu.sync_copy(rows_v, zq_hbm.at[pl.ds(base, TOK_PER_TILE)])
    for t in range(TOK_PER_TILE // SC_LANES):
        ones_v[pl.ds(t * SC_LANES, SC_LANES)] = jnp.ones((SC_LANES,), jnp.float32)
    for t in range(CNT_PER_SUB // SC_LANES):
        zeros_v[pl.ds(t * SC_LANES, SC_LANES)] = jnp.zeros((SC_LANES,), jnp.float32)
    pltpu.sync_copy(zeros_v, shared_counts.at[pl.ds(sid * CNT_PER_SUB, CNT_PER_SUB)])
    plsc.subcore_barrier()
    pltpu.sync_copy(ones_v, shared_counts.at[idx_v], add=True)
    plsc.subcore_barrier()
    @pl.when(sid == 0)
    def _():
        pltpu.sync_copy(shared_counts, counts_hbm.at[cid])


@functools.cache
def _sc_gather_hist():
    return pl.kernel(
        _sc_body,
        out_type=(
            jax.ShapeDtypeStruct((N_TOK, DIM), jnp.float32),
            jax.ShapeDtypeStruct((SC_CORES, N_CODES), jnp.float32),
        ),
        mesh=plsc.VectorSubcoreMesh(core_axis_name="c", subcore_axis_name="s"),
        compiler_params=pltpu.CompilerParams(use_tc_tiling_on_sc=False),
        scratch_types=[
            pltpu.VMEM((TOK_PER_TILE,), jnp.int32),
            pltpu.VMEM((TOK_PER_TILE, DIM), jnp.float32),
            pltpu.VMEM((TOK_PER_TILE,), jnp.float32),
            pltpu.VMEM((CNT_PER_SUB,), jnp.float32),
            pltpu.VMEM_SHARED((N_CODES,), jnp.float32),
            pltpu.SemaphoreType.DMA,
        ],
    )


def _finalize_body(z_ref, zq_ref, c2_ref, loss_ref, perp_ref):
    diff = z_ref[...] - zq_ref[...]
    commitment = jnp.mean(diff * diff)
    loss_ref[...] = jnp.full((1, 1), 0.25 * commitment, jnp.float32)
    counts = c2_ref[0:1, :] + c2_ref[1:2, :]
    avg = counts / float(N_TOK)
    ent = avg * jnp.log(avg + 1e-10)
    perp_ref[...] = jnp.full((1, 1), jnp.exp(-jnp.sum(ent)), jnp.float32)


def _finalize(zf, qf, counts2):
    return pl.pallas_call(
        _finalize_body,
        grid=(1,),
        in_specs=[
            pl.BlockSpec(zf.shape, lambda i: (0, 0)),
            pl.BlockSpec(qf.shape, lambda i: (0, 0)),
            pl.BlockSpec(counts2.shape, lambda i: (0, 0)),
        ],
        out_specs=[
            pl.BlockSpec((1, 1), lambda i: (0, 0)),
            pl.BlockSpec((1, 1), lambda i: (0, 0)),
        ],
        out_shape=[
            jax.ShapeDtypeStruct((1, 1), jnp.float32),
            jax.ShapeDtypeStruct((1, 1), jnp.float32),
        ],
    )(zf, qf, counts2)


def kernel(z, codebook):
    z2d = z.reshape(N_TOK, DIM)
    cbt = codebook.T
    zn2d = jnp.sum(z2d * z2d, axis=1).reshape(N_TOK, 1)
    z2bf = (2.0 * z2d).astype(jnp.bfloat16)
    cbt_bf = cbt.astype(jnp.bfloat16)
    idx2d = _compute_indices(zn2d, z2bf, cbt, cbt_bf)
    idx_flat = idx2d.reshape(N_TOK)
    z_q2d, counts2 = _sc_gather_hist()(codebook, idx_flat)
    zf = z2d.reshape(N_TOK * DIM // 512, 512)
    qf = z_q2d.reshape(N_TOK * DIM // 512, 512)
    loss2d, perp2d = _finalize(zf, qf, counts2)
    z_q = z_q2d.reshape(z.shape)
    z_q_st = z + lax.stop_gradient(z_q - z)
    loss = loss2d.reshape(())
    perplexity = perp2d.reshape(())
    encoding_indices = idx_flat.reshape(z.shape[:-1])
    return (z_q_st, loss, encoding_indices, perplexity)


# speedup vs baseline: 1.4394x; 1.0203x over previous
import functools

import jax
import jax.numpy as jnp
from jax import lax
from jax.experimental import pallas as pl
from jax.experimental.pallas import tpu as pltpu
from jax.experimental.pallas import tpu_sc as plsc

N_CODES = 8192
DIM = 32
N_TOK = 16384
TOK_BLK = 512
CHUNK = 4096

SC_CORES = 2
SC_SUBCORES = 16
SC_LANES = 16
SC_TILES = SC_CORES * SC_SUBCORES
TOK_PER_TILE = N_TOK // SC_TILES
CNT_PER_SUB = N_CODES // SC_SUBCORES


def _argmin_body(zn_ref, z2bf_ref, cbt_ref, cbtbf_ref, idx_ref):
    zn = zn_ref[...]
    zb = z2bf_ref[...]
    acc_v = jnp.zeros((TOK_BLK, 1), jnp.float32)
    acc_i = jnp.zeros((TOK_BLK, 1), jnp.int32)
    for j in range(N_CODES // CHUNK):
        cb = cbt_ref[:, j * CHUNK:(j + 1) * CHUNK]
        cbbf = cbtbf_ref[:, j * CHUNK:(j + 1) * CHUNK]
        cn = jnp.sum(cb * cb, axis=0, keepdims=True)
        zc2 = lax.dot_general(zb, cbbf, (((1,), (0,)), ((), ())),
                              preferred_element_type=jnp.float32)
        d = (zn + cn) - zc2
        m = jnp.min(d, axis=1, keepdims=True)
        gi = lax.broadcasted_iota(jnp.int32, (TOK_BLK, CHUNK), 1)
        cand = jnp.where(d == m, gi, jnp.int32(2**30))
        i = jnp.min(cand, axis=1, keepdims=True) + (j * CHUNK)
        m_bf = m.astype(jnp.bfloat16).astype(jnp.float32)
        if j == 0:
            acc_v, acc_i = m_bf, i
        else:
            repl = m < acc_v
            acc_v = jnp.where(repl, m_bf, acc_v)
            acc_i = jnp.where(repl, i, acc_i)
    idx_ref[...] = acc_i


def _compute_indices(zn2d, z2bf, cbt, cbt_bf):
    return pl.pallas_call(
        _argmin_body,
        grid=(N_TOK // TOK_BLK,),
        in_specs=[
            pl.BlockSpec((TOK_BLK, 1), lambda i: (i, 0)),
            pl.BlockSpec((TOK_BLK, DIM), lambda i: (i, 0)),
            pl.BlockSpec((DIM, N_CODES), lambda i: (0, 0)),
            pl.BlockSpec((DIM, N_CODES), lambda i: (0, 0)),
        ],
        out_specs=pl.BlockSpec((TOK_BLK, 1), lambda i: (i, 0)),
        out_shape=jax.ShapeDtypeStruct((N_TOK, 1), jnp.int32),
    )(zn2d, z2bf, cbt, cbt_bf)


def _sc_body(cb_hbm, idx_hbm, zq_hbm, counts_hbm,
             idx_v, rows_v, ones_v, zeros_v, shared_counts, sem):
    cid = lax.axis_index("c")
    sid = lax.axis_index("s")
    wid = sid * SC_CORES + cid
    base = wid * TOK_PER_TILE
    pltpu.sync_copy(idx_hbm.at[pl.ds(base, TOK_PER_TILE)], idx_v)
    pltpu.async_copy(cb_hbm.at[idx_v], rows_v, sem).wait()
    pltpu.sync_copy(rows_v, zq_hbm.at[pl.ds(base, TOK_PER_TILE)])
    for t in range(TOK_PER_TILE // SC_LANES):
        ones_v[pl.ds(t * SC_LANES, SC_LANES)] = jnp.ones((SC_LANES,), jnp.float32)
    for t in range(CNT_PER_SUB // SC_LANES):
        zeros_v[pl.ds(t * SC_LANES, SC_LANES)] = jnp.zeros((SC_LANES,), jnp.float32)
    pltpu.sync_copy(zeros_v, shared_counts.at[pl.ds(sid * CNT_PER_SUB, CNT_PER_SUB)])
    plsc.subcore_barrier()
    pltpu.sync_copy(ones_v, shared_counts.at[idx_v], add=True)
    plsc.subcore_barrier()
    @pl.when(sid == 0)
    def _():
        pltpu.sync_copy(shared_counts, counts_hbm.at[cid])


@functools.cache
def _sc_gather_hist():
    return pl.kernel(
        _sc_body,
        out_type=(
            jax.ShapeDtypeStruct((N_TOK, DIM), jnp.float32),
            jax.ShapeDtypeStruct((SC_CORES, N_CODES), jnp.float32),
        ),
        mesh=plsc.VectorSubcoreMesh(core_axis_name="c", subcore_axis_name="s"),
        compiler_params=pltpu.CompilerParams(use_tc_tiling_on_sc=False),
        scratch_types=[
            pltpu.VMEM((TOK_PER_TILE,), jnp.int32),
            pltpu.VMEM((TOK_PER_TILE, DIM), jnp.float32),
            pltpu.VMEM((TOK_PER_TILE,), jnp.float32),
            pltpu.VMEM((CNT_PER_SUB,), jnp.float32),
            pltpu.VMEM_SHARED((N_CODES,), jnp.float32),
            pltpu.SemaphoreType.DMA,
        ],
    )


def _finalize_body(z_ref, zq_ref, c2_ref, loss_ref, perp_ref):
    diff = z_ref[...] - zq_ref[...]
    commitment = jnp.mean(diff * diff)
    loss_ref[...] = jnp.full((1, 1), 0.25 * commitment, jnp.float32)
    counts = c2_ref[0:1, :] + c2_ref[1:2, :]
    avg = counts / float(N_TOK)
    ent = avg * jnp.log(avg + 1e-10)
    perp_ref[...] = jnp.full((1, 1), jnp.exp(-jnp.sum(ent)), jnp.float32)


def _finalize(zf, qf, counts2):
    return pl.pallas_call(
        _finalize_body,
        grid=(1,),
        in_specs=[
            pl.BlockSpec(zf.shape, lambda i: (0, 0)),
            pl.BlockSpec(qf.shape, lambda i: (0, 0)),
            pl.BlockSpec(counts2.shape, lambda i: (0, 0)),
        ],
        out_specs=[
            pl.BlockSpec((1, 1), lambda i: (0, 0)),
            pl.BlockSpec((1, 1), lambda i: (0, 0)),
        ],
        out_shape=[
            jax.ShapeDtypeStruct((1, 1), jnp.float32),
            jax.ShapeDtypeStruct((1, 1), jnp.float32),
        ],
    )(zf, qf, counts2)


def kernel(z, codebook):
    z2d = z.reshape(N_TOK, DIM)
    cbt = codebook.T
    zn2d = jnp.sum(z2d * z2d, axis=1).reshape(N_TOK, 1)
    z2bf = (2.0 * z2d).astype(jnp.bfloat16)
    cbt_bf = cbt.astype(jnp.bfloat16)
    idx2d = _compute_indices(zn2d, z2bf, cbt, cbt_bf)
    idx_flat = idx2d.reshape(N_TOK)
    z_q2d, counts2 = _sc_gather_hist()(codebook, idx_flat)
    zf = z2d.reshape(N_TOK * DIM // 512, 512)
    qf = z_q2d.reshape(N_TOK * DIM // 512, 512)
    loss2d, perp2d = _finalize(zf, qf, counts2)
    z_q = z_q2d.reshape(z.shape)
    z_q_st = z + lax.stop_gradient(z_q - z)
    loss = loss2d.reshape(())
    perplexity = perp2d.reshape(())
    encoding_indices = idx_flat.reshape(z.shape[:-1])
    return (z_q_st, loss, encoding_indices, perplexity)


# speedup vs baseline: 1.4402x; 1.0005x over previous
import functools

import jax
import jax.numpy as jnp
from jax import lax
from jax.experimental import pallas as pl
from jax.experimental.pallas import tpu as pltpu
from jax.experimental.pallas import tpu_sc as plsc

N_CODES = 8192
DIM = 32
N_TOK = 16384
TOK_BLK = 512
CHUNK = 4096

SC_CORES = 2
SC_SUBCORES = 16
SC_LANES = 16
SC_TILES = SC_CORES * SC_SUBCORES
TOK_PER_TILE = N_TOK // SC_TILES
CNT_PER_SUB = N_CODES // SC_SUBCORES


def _argmin_body(zn_ref, z2bf_ref, cbt_ref, cbtbf_ref, idx_ref):
    zn = zn_ref[...]
    zb = z2bf_ref[...]
    acc_v = jnp.zeros((TOK_BLK, 1), jnp.float32)
    acc_i = jnp.zeros((TOK_BLK, 1), jnp.int32)
    for j in range(N_CODES // CHUNK):
        cb = cbt_ref[:, j * CHUNK:(j + 1) * CHUNK]
        cbbf = cbtbf_ref[:, j * CHUNK:(j + 1) * CHUNK]
        cn = jnp.sum(cb * cb, axis=0, keepdims=True)
        zc2 = lax.dot_general(zb, cbbf, (((1,), (0,)), ((), ())),
                              preferred_element_type=jnp.float32)
        d = (zn + cn) - zc2
        m = jnp.min(d, axis=1, keepdims=True)
        gi = lax.broadcasted_iota(jnp.int32, (TOK_BLK, CHUNK), 1)
        cand = jnp.where(d == m, gi, jnp.int32(2**30))
        i = jnp.min(cand, axis=1, keepdims=True) + (j * CHUNK)
        m_bf = m.astype(jnp.bfloat16).astype(jnp.float32)
        if j == 0:
            acc_v, acc_i = m_bf, i
        else:
            repl = m < acc_v
            acc_v = jnp.where(repl, m_bf, acc_v)
            acc_i = jnp.where(repl, i, acc_i)
    idx_ref[...] = acc_i


def _compute_indices(zn2d, z2bf, cbt, cbt_bf):
    return pl.pallas_call(
        _argmin_body,
        grid=(N_TOK // TOK_BLK,),
        in_specs=[
            pl.BlockSpec((TOK_BLK, 1), lambda i: (i, 0)),
            pl.BlockSpec((TOK_BLK, DIM), lambda i: (i, 0)),
            pl.BlockSpec((DIM, N_CODES), lambda i: (0, 0)),
            pl.BlockSpec((DIM, N_CODES), lambda i: (0, 0)),
        ],
        out_specs=pl.BlockSpec((TOK_BLK, 1), lambda i: (i, 0)),
        out_shape=jax.ShapeDtypeStruct((N_TOK, 1), jnp.int32),
        compiler_params=pltpu.CompilerParams(
            dimension_semantics=("parallel",)),
    )(zn2d, z2bf, cbt, cbt_bf)


def _sc_body(cb_hbm, idx_hbm, zq_hbm, counts_hbm,
             idx_v, rows_v, ones_v, zeros_v, shared_counts, sem):
    cid = lax.axis_index("c")
    sid = lax.axis_index("s")
    wid = sid * SC_CORES + cid
    base = wid * TOK_PER_TILE
    pltpu.sync_copy(idx_hbm.at[pl.ds(base, TOK_PER_TILE)], idx_v)
    pltpu.async_copy(cb_hbm.at[idx_v], rows_v, sem).wait()
    pltpu.sync_copy(rows_v, zq_hbm.at[pl.ds(base, TOK_PER_TILE)])
    for t in range(TOK_PER_TILE // SC_LANES):
        ones_v[pl.ds(t * SC_LANES, SC_LANES)] = jnp.ones((SC_LANES,), jnp.float32)
    for t in range(CNT_PER_SUB // SC_LANES):
        zeros_v[pl.ds(t * SC_LANES, SC_LANES)] = jnp.zeros((SC_LANES,), jnp.float32)
    pltpu.sync_copy(zeros_v, shared_counts.at[pl.ds(sid * CNT_PER_SUB, CNT_PER_SUB)])
    plsc.subcore_barrier()
    pltpu.sync_copy(ones_v, shared_counts.at[idx_v], add=True)
    plsc.subcore_barrier()
    @pl.when(sid == 0)
    def _():
        pltpu.sync_copy(shared_counts, counts_hbm.at[cid])


@functools.cache
def _sc_gather_hist():
    return pl.kernel(
        _sc_body,
        out_type=(
            jax.ShapeDtypeStruct((N_TOK, DIM), jnp.float32),
            jax.ShapeDtypeStruct((SC_CORES, N_CODES), jnp.float32),
        ),
        mesh=plsc.VectorSubcoreMesh(core_axis_name="c", subcore_axis_name="s"),
        compiler_params=pltpu.CompilerParams(use_tc_tiling_on_sc=False),
        scratch_types=[
            pltpu.VMEM((TOK_PER_TILE,), jnp.int32),
            pltpu.VMEM((TOK_PER_TILE, DIM), jnp.float32),
            pltpu.VMEM((TOK_PER_TILE,), jnp.float32),
            pltpu.VMEM((CNT_PER_SUB,), jnp.float32),
            pltpu.VMEM_SHARED((N_CODES,), jnp.float32),
            pltpu.SemaphoreType.DMA,
        ],
    )


def _finalize_body(z_ref, zq_ref, c2_ref, loss_ref, perp_ref):
    diff = z_ref[...] - zq_ref[...]
    commitment = jnp.mean(diff * diff)
    loss_ref[...] = jnp.full((1, 1), 0.25 * commitment, jnp.float32)
    counts = c2_ref[0:1, :] + c2_ref[1:2, :]
    avg = counts / float(N_TOK)
    ent = avg * jnp.log(avg + 1e-10)
    perp_ref[...] = jnp.full((1, 1), jnp.exp(-jnp.sum(ent)), jnp.float32)


def _finalize(zf, qf, counts2):
    return pl.pallas_call(
        _finalize_body,
        grid=(1,),
        in_specs=[
            pl.BlockSpec(zf.shape, lambda i: (0, 0)),
            pl.BlockSpec(qf.shape, lambda i: (0, 0)),
            pl.BlockSpec(counts2.shape, lambda i: (0, 0)),
        ],
        out_specs=[
            pl.BlockSpec((1, 1), lambda i: (0, 0)),
            pl.BlockSpec((1, 1), lambda i: (0, 0)),
        ],
        out_shape=[
            jax.ShapeDtypeStruct((1, 1), jnp.float32),
            jax.ShapeDtypeStruct((1, 1), jnp.float32),
        ],
    )(zf, qf, counts2)


def kernel(z, codebook):
    z2d = z.reshape(N_TOK, DIM)
    cbt = codebook.T
    zn2d = jnp.sum(z2d * z2d, axis=1).reshape(N_TOK, 1)
    z2bf = (2.0 * z2d).astype(jnp.bfloat16)
    cbt_bf = cbt.astype(jnp.bfloat16)
    idx2d = _compute_indices(zn2d, z2bf, cbt, cbt_bf)
    idx_flat = idx2d.reshape(N_TOK)
    z_q2d, counts2 = _sc_gather_hist()(codebook, idx_flat)
    zf = z2d.reshape(N_TOK * DIM // 512, 512)
    qf = z_q2d.reshape(N_TOK * DIM // 512, 512)
    loss2d, perp2d = _finalize(zf, qf, counts2)
    z_q = z_q2d.reshape(z.shape)
    z_q_st = z + lax.stop_gradient(z_q - z)
    loss = loss2d.reshape(())
    perplexity = perp2d.reshape(())
    encoding_indices = idx_flat.reshape(z.shape[:-1])
    return (z_q_st, loss, encoding_indices, perplexity)
